# baseline (speedup 1.0000x reference)
import functools

import jax
import jax.numpy as jnp
import numpy as np
from jax import lax
from jax.experimental import pallas as pl
from jax.experimental.pallas import tpu as pltpu
from jax.experimental.pallas import tpu_sc as plsc

_B, _N, _C, _K = 4, 4096, 128, 16
_EPS = 1e-5
_SCALE = float(np.sqrt(_C))

_BQ = 512
_BN = 256


def _prep_body(xytp_ref, feat_ref, wpos_ref, bpos_ref, wlt_ref, blt_ref,
               st_ref, q_ref, pb_ref):
    x = xytp_ref[0]
    f = feat_ref[0]
    P = jnp.dot(x, wpos_ref[...], preferred_element_type=jnp.float32,
                precision=lax.Precision.HIGHEST)
    lt = jnp.dot(f, wlt_ref[...], preferred_element_type=jnp.float32,
                 precision=lax.Precision.HIGHEST) + blt_ref[...]
    varphi = lt[:, :_C]
    psi = lt[:, _C:2 * _C]
    alpha = lt[:, 2 * _C:]
    Pb = P + bpos_ref[...]
    st_ref[0, :, :_C] = psi + P
    st_ref[0, :, _C:] = alpha - P
    q_ref[0] = varphi + Pb
    pb_ref[0] = Pb


def _prep(xytp, features, W_pos, b_pos, W_lt, b_lt):
    grid = (_B, _N // _BQ)
    return pl.pallas_call(
        _prep_body,
        grid=grid,
        in_specs=[
            pl.BlockSpec((1, _BQ, 4), lambda b, i: (b, i, 0)),
            pl.BlockSpec((1, _BQ, _C), lambda b, i: (b, i, 0)),
            pl.BlockSpec((4, _C), lambda b, i: (0, 0)),
            pl.BlockSpec((1, _C), lambda b, i: (0, 0)),
            pl.BlockSpec((_C, 3 * _C), lambda b, i: (0, 0)),
            pl.BlockSpec((1, 3 * _C), lambda b, i: (0, 0)),
        ],
        out_specs=[
            pl.BlockSpec((1, _BQ, 2 * _C), lambda b, i: (b, i, 0)),
            pl.BlockSpec((1, _BQ, _C), lambda b, i: (b, i, 0)),
            pl.BlockSpec((1, _BQ, _C), lambda b, i: (b, i, 0)),
        ],
        out_shape=[
            jax.ShapeDtypeStruct((_B, _N, 2 * _C), jnp.float32),
            jax.ShapeDtypeStruct((_B, _N, _C), jnp.float32),
            jax.ShapeDtypeStruct((_B, _N, _C), jnp.float32),
        ],
    )(xytp, features, W_pos, b_pos.reshape(1, _C), W_lt, b_lt.reshape(1, 3 * _C))


def _knn_body(xq_ref, xa_ref, idx_ref):
    b = pl.program_id(0)
    lane4 = lax.broadcasted_iota(jnp.int32, (1, 4), 1)
    xq = jnp.where(lane4 < 3, xq_ref[0], 0.0)
    xa = jnp.where(lane4 < 3, xa_ref[0], 0.0)
    sqq = jnp.sum(xq * xq, axis=1, keepdims=True)
    ones14 = jnp.full((1, 4), 1.0, jnp.float32)
    sqa_row = lax.dot_general(ones14, xa * xa, (((1,), (1,)), ((), ())),
                              preferred_element_type=jnp.float32,
                              precision=lax.Precision.HIGHEST)
    dot = lax.dot_general(xq, xa, (((1,), (1,)), ((), ())),
                          preferred_element_type=jnp.float32,
                          precision=lax.Precision.DEFAULT)
    d2 = (sqq + sqa_row) - 2.0 * dot

    kbits = lax.bitcast_convert_type(d2, jnp.int32)
    kmono = kbits ^ ((kbits >> 31) & jnp.int32(0x7FFFFFFF))
    nchunks = _N // 128
    v = [(kmono[:, c * 128:(c + 1) * 128] & jnp.int32(~0x1F)) | jnp.int32(c)
         for c in range(nchunks)]

    def _ce(a, b):
        return jnp.minimum(a, b), jnp.maximum(a, b)

    def _bitonic_sort16(w):
        n = 16
        k = 2
        while k <= n:
            j = k // 2
            while j >= 1:
                for i in range(n):
                    l = i ^ j
                    if l > i:
                        lo, hi = _ce(w[i], w[l])
                        if (i & k) == 0:
                            w[i], w[l] = lo, hi
                        else:
                            w[i], w[l] = hi, lo
                j //= 2
            k *= 2
        return w

    def _bitonic_merge16(w):
        for j in (8, 4, 2, 1):
            for i in range(16):
                l = i ^ j
                if l > i:
                    w[i], w[l] = _ce(w[i], w[l])
        return w

    a = _bitonic_sort16(v[:16])
    bb = _bitonic_sort16(v[16:])
    t16 = [jnp.minimum(a[i], bb[15 - i]) for i in range(16)]
    L = _bitonic_merge16(t16)

    lane = lax.broadcasted_iota(jnp.int32, (_BQ, 128), 1)
    big = jnp.int32(2 ** 30)
    maxi = jnp.int32(0x7FFFFFFF)
    for t in range(_K):
        m = jnp.min(L[0], axis=1, keepdims=True)
        cand = jnp.where(L[0] == m, lane, big)
        l = jnp.min(cand, axis=1, keepdims=True)
        idx_ref[0, :, t:t + 1] = (m & 31) * 128 + l + b * _N
        if t < _K - 1:
            sel = lane == l
            for p in range(15):
                L[p] = jnp.where(sel, L[p + 1], L[p])
            L[15] = jnp.where(sel, maxi, L[15])
    return


def _knn(xytp):
    grid = (_B, _N // _BQ)
    return pl.pallas_call(
        _knn_body,
        grid=grid,
        in_specs=[
            pl.BlockSpec((1, _BQ, 4), lambda b, i: (b, i, 0)),
            pl.BlockSpec((1, _N, 4), lambda b, i: (b, 0, 0)),
        ],
        out_specs=pl.BlockSpec((1, _BQ, _K), lambda b, i: (b, i, 0)),
        out_shape=jax.ShapeDtypeStruct((_B, _N, _K), jnp.int32),
    )(xytp, xytp)


def _sc_gather(table, idxg):
    R, D = table.shape
    M = idxg.shape[0]
    NW = 32
    per_w = M // NW
    CH = 128
    mesh = plsc.VectorSubcoreMesh(core_axis_name="c", subcore_axis_name="s")

    @functools.partial(
        pl.kernel, mesh=mesh,
        out_type=jax.ShapeDtypeStruct((M, D), jnp.float32),
        scratch_types=[
            pltpu.VMEM((CH,), jnp.int32),
            pltpu.VMEM((CH, D), jnp.float32),
            pltpu.SemaphoreType.DMA,
        ],
    )
    def gather_k(tab_hbm, idx_hbm, out_hbm, idx_v, rows_v, sem):
        c = lax.axis_index("c")
        s = lax.axis_index("s")
        wid = s * 2 + c
        base = wid * per_w

        def body(i, carry):
            off = base + i * CH
            pltpu.sync_copy(idx_hbm.at[pl.ds(off, CH)], idx_v)
            pltpu.async_copy(tab_hbm.at[idx_v], rows_v, sem).wait()
            pltpu.sync_copy(rows_v, out_hbm.at[pl.ds(off, CH)])
            return carry

        lax.fori_loop(0, per_w // CH, body, 0)

    return gather_k(table, idxg)


def _attn_body(q_ref, pb_ref, g_ref, gamma_ref, beta_ref, o_ref):
    Q = q_ref[0]
    Pb = pb_ref[0]
    G = g_ref[0].reshape(_BN, _K, 2 * _C)
    S = G[:, :, :_C]
    T = G[:, :, _C:]
    pre = Q[:, None, :] - S
    mu = jnp.mean(pre, axis=2, keepdims=True)
    d = pre - mu
    var = jnp.mean(d * d, axis=2, keepdims=True)
    r = 1.0 / jnp.sqrt(var + _EPS)
    ln = d * r * gamma_ref[...] + beta_ref[...]
    z = ln * jnp.float32(1.0 / _SCALE)
    zm = jnp.max(z, axis=1, keepdims=True)
    e = jnp.exp(z - zm)
    w = e * (1.0 / jnp.sum(e, axis=1, keepdims=True))
    o_ref[0] = Pb + jnp.sum(w * T, axis=1)


def _attention(Q, Pb, G, gamma, beta):
    grid = (_B, _N // _BN)
    return pl.pallas_call(
        _attn_body,
        grid=grid,
        in_specs=[
            pl.BlockSpec((1, _BN, _C), lambda b, i: (b, i, 0)),
            pl.BlockSpec((1, _BN, _C), lambda b, i: (b, i, 0)),
            pl.BlockSpec((1, _BN * _K, 2 * _C), lambda b, i: (b, i, 0)),
            pl.BlockSpec((1, _C), lambda b, i: (0, 0)),
            pl.BlockSpec((1, _C), lambda b, i: (0, 0)),
        ],
        out_specs=pl.BlockSpec((1, _BN, _C), lambda b, i: (b, i, 0)),
        out_shape=jax.ShapeDtypeStruct((_B, _N, _C), jnp.float32),
    )(Q, Pb, G, gamma.reshape(1, _C), beta.reshape(1, _C))


def kernel(xytp, features, W_pos, b_pos, W_lt, b_lt, gamma, beta):
    ST, Q, Pb = _prep(xytp, features, W_pos, b_pos, W_lt, b_lt)
    idxg = _knn(xytp)
    G = _sc_gather(ST.reshape(_B * _N, 2 * _C), idxg.reshape(_B * _N * _K))
    G = G.reshape(_B, _N * _K, 2 * _C)
    return _attention(Q, Pb, G, gamma, beta)


# speedup vs baseline: 1.1812x; 1.1812x over previous
import functools

import jax
import jax.numpy as jnp
import numpy as np
from jax import lax
from jax.experimental import pallas as pl
from jax.experimental.pallas import tpu as pltpu
from jax.experimental.pallas import tpu_sc as plsc

_B, _N, _C, _K = 4, 4096, 128, 16
_EPS = 1e-5
_SCALE = float(np.sqrt(_C))

_BQ = 512
_BN = 256


def _prep_body(xytp_ref, feat_ref, wpos_ref, bpos_ref, wlt_ref, blt_ref,
               st_ref, q_ref, pb_ref):
    x = xytp_ref[0]
    f = feat_ref[0]
    P = jnp.dot(x, wpos_ref[...], preferred_element_type=jnp.float32,
                precision=lax.Precision.HIGHEST)
    lt = jnp.dot(f, wlt_ref[...], preferred_element_type=jnp.float32,
                 precision=lax.Precision.HIGHEST) + blt_ref[...]
    varphi = lt[:, :_C]
    psi = lt[:, _C:2 * _C]
    alpha = lt[:, 2 * _C:]
    Pb = P + bpos_ref[...]
    st_ref[0, :, :_C] = psi + P
    st_ref[0, :, _C:] = alpha - P
    q_ref[0] = varphi + Pb
    pb_ref[0] = Pb


def _prep(xytp, features, W_pos, b_pos, W_lt, b_lt):
    grid = (_B, _N // _BQ)
    return pl.pallas_call(
        _prep_body,
        grid=grid,
        in_specs=[
            pl.BlockSpec((1, _BQ, 4), lambda b, i: (b, i, 0)),
            pl.BlockSpec((1, _BQ, _C), lambda b, i: (b, i, 0)),
            pl.BlockSpec((4, _C), lambda b, i: (0, 0)),
            pl.BlockSpec((1, _C), lambda b, i: (0, 0)),
            pl.BlockSpec((_C, 3 * _C), lambda b, i: (0, 0)),
            pl.BlockSpec((1, 3 * _C), lambda b, i: (0, 0)),
        ],
        out_specs=[
            pl.BlockSpec((1, _BQ, 2 * _C), lambda b, i: (b, i, 0)),
            pl.BlockSpec((1, _BQ, _C), lambda b, i: (b, i, 0)),
            pl.BlockSpec((1, _BQ, _C), lambda b, i: (b, i, 0)),
        ],
        out_shape=[
            jax.ShapeDtypeStruct((_B, _N, 2 * _C), jnp.float32),
            jax.ShapeDtypeStruct((_B, _N, _C), jnp.float32),
            jax.ShapeDtypeStruct((_B, _N, _C), jnp.float32),
        ],
    )(xytp, features, W_pos, b_pos.reshape(1, _C), W_lt, b_lt.reshape(1, 3 * _C))


def _knn_body(xq_ref, xa_ref, idx_ref, *, base):
    lane4 = lax.broadcasted_iota(jnp.int32, (1, 4), 1)
    xq = jnp.where(lane4 < 3, xq_ref[0], 0.0)
    xa = jnp.where(lane4 < 3, xa_ref[0], 0.0)
    sqq = jnp.sum(xq * xq, axis=1, keepdims=True)
    ones14 = jnp.full((1, 4), 1.0, jnp.float32)
    sqa_row = lax.dot_general(ones14, xa * xa, (((1,), (1,)), ((), ())),
                              preferred_element_type=jnp.float32,
                              precision=lax.Precision.HIGHEST)
    dot = lax.dot_general(xq, xa, (((1,), (1,)), ((), ())),
                          preferred_element_type=jnp.float32,
                          precision=lax.Precision.DEFAULT)
    d2 = (sqq + sqa_row) - 2.0 * dot

    kbits = lax.bitcast_convert_type(d2, jnp.int32)
    kmono = kbits ^ ((kbits >> 31) & jnp.int32(0x7FFFFFFF))
    nchunks = _N // 128
    v = [(kmono[:, c * 128:(c + 1) * 128] & jnp.int32(~0x1F)) | jnp.int32(c)
         for c in range(nchunks)]

    def _ce(a, b):
        return jnp.minimum(a, b), jnp.maximum(a, b)

    def _bitonic_sort16(w):
        n = 16
        k = 2
        while k <= n:
            j = k // 2
            while j >= 1:
                for i in range(n):
                    l = i ^ j
                    if l > i:
                        lo, hi = _ce(w[i], w[l])
                        if (i & k) == 0:
                            w[i], w[l] = lo, hi
                        else:
                            w[i], w[l] = hi, lo
                j //= 2
            k *= 2
        return w

    def _bitonic_merge16(w):
        for j in (8, 4, 2, 1):
            for i in range(16):
                l = i ^ j
                if l > i:
                    w[i], w[l] = _ce(w[i], w[l])
        return w

    a = _bitonic_sort16(v[:16])
    bb = _bitonic_sort16(v[16:])
    t16 = [jnp.minimum(a[i], bb[15 - i]) for i in range(16)]
    L = _bitonic_merge16(t16)

    lane = lax.broadcasted_iota(jnp.int32, (_BQ, 128), 1)
    big = jnp.int32(2 ** 30)
    maxi = jnp.int32(0x7FFFFFFF)
    for t in range(_K):
        m = jnp.min(L[0], axis=1, keepdims=True)
        cand = jnp.where(L[0] == m, lane, big)
        l = jnp.min(cand, axis=1, keepdims=True)
        idx_ref[0, :, t:t + 1] = (m & 31) * 128 + l + base
        if t < _K - 1:
            sel = lane == l
            for p in range(15):
                L[p] = jnp.where(sel, L[p + 1], L[p])
            L[15] = jnp.where(sel, maxi, L[15])
    return


def _knn(x_b, base):
    grid = (_N // _BQ,)
    return pl.pallas_call(
        functools.partial(_knn_body, base=base),
        grid=grid,
        in_specs=[
            pl.BlockSpec((1, _BQ, 4), lambda i: (0, i, 0)),
            pl.BlockSpec((1, _N, 4), lambda i: (0, 0, 0)),
        ],
        out_specs=pl.BlockSpec((1, _BQ, _K), lambda i: (0, i, 0)),
        out_shape=jax.ShapeDtypeStruct((1, _N, _K), jnp.int32),
    )(x_b, x_b)


def _sc_gather(table, idxg):
    R, D = table.shape
    M = idxg.shape[0]
    NW = 32
    per_w = M // NW
    CH = 128
    mesh = plsc.VectorSubcoreMesh(core_axis_name="c", subcore_axis_name="s")

    @functools.partial(
        pl.kernel, mesh=mesh,
        out_type=jax.ShapeDtypeStruct((M, D), jnp.float32),
        scratch_types=[
            pltpu.VMEM((CH,), jnp.int32),
            pltpu.VMEM((CH, D), jnp.float32),
            pltpu.SemaphoreType.DMA,
        ],
    )
    def gather_k(tab_hbm, idx_hbm, out_hbm, idx_v, rows_v, sem):
        c = lax.axis_index("c")
        s = lax.axis_index("s")
        wid = s * 2 + c
        base = wid * per_w

        def body(i, carry):
            off = base + i * CH
            pltpu.sync_copy(idx_hbm.at[pl.ds(off, CH)], idx_v)
            pltpu.async_copy(tab_hbm.at[idx_v], rows_v, sem).wait()
            pltpu.sync_copy(rows_v, out_hbm.at[pl.ds(off, CH)])
            return carry

        lax.fori_loop(0, per_w // CH, body, 0)

    return gather_k(table, idxg)


def _attn_body(q_ref, pb_ref, g_ref, gamma_ref, beta_ref, o_ref):
    Q = q_ref[0]
    Pb = pb_ref[0]
    G = g_ref[0].reshape(_BN, _K, 2 * _C)
    S = G[:, :, :_C]
    T = G[:, :, _C:]
    pre = Q[:, None, :] - S
    mu = jnp.mean(pre, axis=2, keepdims=True)
    d = pre - mu
    var = jnp.mean(d * d, axis=2, keepdims=True)
    r = 1.0 / jnp.sqrt(var + _EPS)
    ln = d * r * gamma_ref[...] + beta_ref[...]
    z = ln * jnp.float32(1.0 / _SCALE)
    zm = jnp.max(z, axis=1, keepdims=True)
    e = jnp.exp(z - zm)
    w = e * (1.0 / jnp.sum(e, axis=1, keepdims=True))
    o_ref[0] = Pb + jnp.sum(w * T, axis=1)


def _attention(Q, Pb, G, gamma, beta):
    grid = (_N // _BN,)
    return pl.pallas_call(
        _attn_body,
        grid=grid,
        in_specs=[
            pl.BlockSpec((1, _BN, _C), lambda i: (0, i, 0)),
            pl.BlockSpec((1, _BN, _C), lambda i: (0, i, 0)),
            pl.BlockSpec((1, _BN * _K, 2 * _C), lambda i: (0, i, 0)),
            pl.BlockSpec((1, _C), lambda i: (0, 0)),
            pl.BlockSpec((1, _C), lambda i: (0, 0)),
        ],
        out_specs=pl.BlockSpec((1, _BN, _C), lambda i: (0, i, 0)),
        out_shape=jax.ShapeDtypeStruct((1, _N, _C), jnp.float32),
    )(Q, Pb, G, gamma.reshape(1, _C), beta.reshape(1, _C))


def kernel(xytp, features, W_pos, b_pos, W_lt, b_lt, gamma, beta):
    ST, Q, Pb = _prep(xytp, features, W_pos, b_pos, W_lt, b_lt)
    ST_flat = ST.reshape(_B * _N, 2 * _C)
    outs = []
    for b in range(_B):
        idx_b = _knn(lax.slice_in_dim(xytp, b, b + 1, axis=0), b * _N)
        G_b = _sc_gather(ST_flat, idx_b.reshape(_N * _K))
        outs.append(_attention(
            lax.slice_in_dim(Q, b, b + 1, axis=0),
            lax.slice_in_dim(Pb, b, b + 1, axis=0),
            G_b.reshape(1, _N * _K, 2 * _C), gamma, beta))
    return jnp.concatenate(outs, axis=0)


# speedup vs baseline: 1.1972x; 1.0136x over previous
import functools

import jax
import jax.numpy as jnp
import numpy as np
from jax import lax
from jax.experimental import pallas as pl
from jax.experimental.pallas import tpu as pltpu
from jax.experimental.pallas import tpu_sc as plsc

_B, _N, _C, _K = 4, 4096, 128, 16
_EPS = 1e-5
_SCALE = float(np.sqrt(_C))

_BQ = 512
_BN = 256


def _prep_body(xytp_ref, feat_ref, wpos_ref, bpos_ref, wlt_ref, blt_ref,
               st_ref, q_ref, pb_ref):
    x = xytp_ref[0]
    f = feat_ref[0]
    P = jnp.dot(x, wpos_ref[...], preferred_element_type=jnp.float32,
                precision=lax.Precision.HIGHEST)
    lt = jnp.dot(f, wlt_ref[...], preferred_element_type=jnp.float32,
                 precision=lax.Precision.HIGHEST) + blt_ref[...]
    varphi = lt[:, :_C]
    psi = lt[:, _C:2 * _C]
    alpha = lt[:, 2 * _C:]
    Pb = P + bpos_ref[...]
    sbits = lax.bitcast_convert_type(psi + P, jnp.int32)
    tbits = lax.bitcast_convert_type(alpha - P, jnp.int32)
    half = jnp.int32(0x8000)
    st_ref[0] = (((sbits + half) >> 16) & jnp.int32(0xFFFF)) | \
                ((tbits + half) & jnp.int32(-65536))
    q_ref[0] = varphi + Pb
    pb_ref[0] = Pb


def _prep(xytp, features, W_pos, b_pos, W_lt, b_lt):
    grid = (_B, _N // _BQ)
    return pl.pallas_call(
        _prep_body,
        grid=grid,
        in_specs=[
            pl.BlockSpec((1, _BQ, 4), lambda b, i: (b, i, 0)),
            pl.BlockSpec((1, _BQ, _C), lambda b, i: (b, i, 0)),
            pl.BlockSpec((4, _C), lambda b, i: (0, 0)),
            pl.BlockSpec((1, _C), lambda b, i: (0, 0)),
            pl.BlockSpec((_C, 3 * _C), lambda b, i: (0, 0)),
            pl.BlockSpec((1, 3 * _C), lambda b, i: (0, 0)),
        ],
        out_specs=[
            pl.BlockSpec((1, _BQ, _C), lambda b, i: (b, i, 0)),
            pl.BlockSpec((1, _BQ, _C), lambda b, i: (b, i, 0)),
            pl.BlockSpec((1, _BQ, _C), lambda b, i: (b, i, 0)),
        ],
        out_shape=[
            jax.ShapeDtypeStruct((_B, _N, _C), jnp.int32),
            jax.ShapeDtypeStruct((_B, _N, _C), jnp.float32),
            jax.ShapeDtypeStruct((_B, _N, _C), jnp.float32),
        ],
    )(xytp, features, W_pos, b_pos.reshape(1, _C), W_lt, b_lt.reshape(1, 3 * _C))


def _knn_body(xq_ref, xa_ref, idx_ref, *, base):
    lane4 = lax.broadcasted_iota(jnp.int32, (1, 4), 1)
    xq = jnp.where(lane4 < 3, xq_ref[0], 0.0)
    xa = jnp.where(lane4 < 3, xa_ref[0], 0.0)
    sqq = jnp.sum(xq * xq, axis=1, keepdims=True)
    ones14 = jnp.full((1, 4), 1.0, jnp.float32)
    sqa_row = lax.dot_general(ones14, xa * xa, (((1,), (1,)), ((), ())),
                              preferred_element_type=jnp.float32,
                              precision=lax.Precision.HIGHEST)
    dot = lax.dot_general(xq, xa, (((1,), (1,)), ((), ())),
                          preferred_element_type=jnp.float32,
                          precision=lax.Precision.DEFAULT)
    d2 = (sqq + sqa_row) - 2.0 * dot

    kbits = lax.bitcast_convert_type(d2, jnp.int32)
    kmono = kbits ^ ((kbits >> 31) & jnp.int32(0x7FFFFFFF))
    nchunks = _N // 128
    v = [(kmono[:, c * 128:(c + 1) * 128] & jnp.int32(~0x1F)) | jnp.int32(c)
         for c in range(nchunks)]

    def _ce(a, b):
        return jnp.minimum(a, b), jnp.maximum(a, b)

    def _bitonic_sort16(w):
        n = 16
        k = 2
        while k <= n:
            j = k // 2
            while j >= 1:
                for i in range(n):
                    l = i ^ j
                    if l > i:
                        lo, hi = _ce(w[i], w[l])
                        if (i & k) == 0:
                            w[i], w[l] = lo, hi
                        else:
                            w[i], w[l] = hi, lo
                j //= 2
            k *= 2
        return w

    def _bitonic_merge16(w):
        for j in (8, 4, 2, 1):
            for i in range(16):
                l = i ^ j
                if l > i:
                    w[i], w[l] = _ce(w[i], w[l])
        return w

    a = _bitonic_sort16(v[:16])
    bb = _bitonic_sort16(v[16:])
    t16 = [jnp.minimum(a[i], bb[15 - i]) for i in range(16)]
    L = _bitonic_merge16(t16)

    lane = lax.broadcasted_iota(jnp.int32, (_BQ, 128), 1)
    big = jnp.int32(2 ** 30)
    maxi = jnp.int32(0x7FFFFFFF)
    for t in range(_K):
        m = jnp.min(L[0], axis=1, keepdims=True)
        cand = jnp.where(L[0] == m, lane, big)
        l = jnp.min(cand, axis=1, keepdims=True)
        idx_ref[0, :, t:t + 1] = (m & 31) * 128 + l + base
        if t < _K - 1:
            sel = lane == l
            for p in range(15):
                L[p] = jnp.where(sel, L[p + 1], L[p])
            L[15] = jnp.where(sel, maxi, L[15])
    return


def _knn(x_b, base):
    grid = (_N // _BQ,)
    return pl.pallas_call(
        functools.partial(_knn_body, base=base),
        grid=grid,
        in_specs=[
            pl.BlockSpec((1, _BQ, 4), lambda i: (0, i, 0)),
            pl.BlockSpec((1, _N, 4), lambda i: (0, 0, 0)),
        ],
        out_specs=pl.BlockSpec((1, _BQ, _K), lambda i: (0, i, 0)),
        out_shape=jax.ShapeDtypeStruct((1, _N, _K), jnp.int32),
    )(x_b, x_b)


def _sc_gather(table, idxg):
    R, D = table.shape
    M = idxg.shape[0]
    NW = 32
    per_w = M // NW
    CH = 128
    mesh = plsc.VectorSubcoreMesh(core_axis_name="c", subcore_axis_name="s")

    @functools.partial(
        pl.kernel, mesh=mesh,
        out_type=jax.ShapeDtypeStruct((M, D), table.dtype),
        scratch_types=[
            pltpu.VMEM((CH,), jnp.int32),
            pltpu.VMEM((CH, D), table.dtype),
            pltpu.SemaphoreType.DMA,
        ],
    )
    def gather_k(tab_hbm, idx_hbm, out_hbm, idx_v, rows_v, sem):
        c = lax.axis_index("c")
        s = lax.axis_index("s")
        wid = s * 2 + c
        base = wid * per_w

        def body(i, carry):
            off = base + i * CH
            pltpu.sync_copy(idx_hbm.at[pl.ds(off, CH)], idx_v)
            pltpu.async_copy(tab_hbm.at[idx_v], rows_v, sem).wait()
            pltpu.sync_copy(rows_v, out_hbm.at[pl.ds(off, CH)])
            return carry

        lax.fori_loop(0, per_w // CH, body, 0)

    return gather_k(table, idxg)


def _attn_body(q_ref, pb_ref, g_ref, gamma_ref, beta_ref, o_ref):
    Q = q_ref[0]
    Pb = pb_ref[0]
    G = g_ref[0].reshape(_BN, _K, _C)
    S = lax.bitcast_convert_type(G << 16, jnp.float32)
    T = lax.bitcast_convert_type(G & jnp.int32(-65536), jnp.float32)
    pre = Q[:, None, :] - S
    mu = jnp.mean(pre, axis=2, keepdims=True)
    d = pre - mu
    var = jnp.mean(d * d, axis=2, keepdims=True)
    r = 1.0 / jnp.sqrt(var + _EPS)
    ln = d * r * gamma_ref[...] + beta_ref[...]
    z = ln * jnp.float32(1.0 / _SCALE)
    zm = jnp.max(z, axis=1, keepdims=True)
    e = jnp.exp(z - zm)
    w = e * (1.0 / jnp.sum(e, axis=1, keepdims=True))
    o_ref[0] = Pb + jnp.sum(w * T, axis=1)


def _attention(Q, Pb, G, gamma, beta):
    grid = (_N // _BN,)
    return pl.pallas_call(
        _attn_body,
        grid=grid,
        in_specs=[
            pl.BlockSpec((1, _BN, _C), lambda i: (0, i, 0)),
            pl.BlockSpec((1, _BN, _C), lambda i: (0, i, 0)),
            pl.BlockSpec((1, _BN * _K, _C), lambda i: (0, i, 0)),
            pl.BlockSpec((1, _C), lambda i: (0, 0)),
            pl.BlockSpec((1, _C), lambda i: (0, 0)),
        ],
        out_specs=pl.BlockSpec((1, _BN, _C), lambda i: (0, i, 0)),
        out_shape=jax.ShapeDtypeStruct((1, _N, _C), jnp.float32),
    )(Q, Pb, G, gamma.reshape(1, _C), beta.reshape(1, _C))


def kernel(xytp, features, W_pos, b_pos, W_lt, b_lt, gamma, beta):
    ST, Q, Pb = _prep(xytp, features, W_pos, b_pos, W_lt, b_lt)
    ST_flat = ST.reshape(_B * _N, _C)
    outs = []
    for b in range(_B):
        idx_b = _knn(lax.slice_in_dim(xytp, b, b + 1, axis=0), b * _N)
        G_b = _sc_gather(ST_flat, idx_b.reshape(_N * _K))
        outs.append(_attention(
            lax.slice_in_dim(Q, b, b + 1, axis=0),
            lax.slice_in_dim(Pb, b, b + 1, axis=0),
            G_b.reshape(1, _N * _K, _C), gamma, beta))
    return jnp.concatenate(outs, axis=0)


# speedup vs baseline: 1.2411x; 1.0367x over previous
import functools

import jax
import jax.numpy as jnp
import numpy as np
from jax import lax
from jax.experimental import pallas as pl
from jax.experimental.pallas import tpu as pltpu
from jax.experimental.pallas import tpu_sc as plsc

_B, _N, _C, _K = 4, 4096, 128, 16
_EPS = 1e-5
_SCALE = float(np.sqrt(_C))

_BQ = 1024
_BN = 256


def _prep_body(xytp_ref, feat_ref, wpos_ref, bpos_ref, wlt_ref, blt_ref,
               st_ref, q_ref, pb_ref):
    x = xytp_ref[0]
    f = feat_ref[0]
    P = jnp.dot(x, wpos_ref[...], preferred_element_type=jnp.float32,
                precision=lax.Precision.HIGHEST)
    lt = jnp.dot(f, wlt_ref[...], preferred_element_type=jnp.float32,
                 precision=lax.Precision.HIGHEST) + blt_ref[...]
    varphi = lt[:, :_C]
    psi = lt[:, _C:2 * _C]
    alpha = lt[:, 2 * _C:]
    Pb = P + bpos_ref[...]
    sbits = lax.bitcast_convert_type(psi + P, jnp.int32)
    tbits = lax.bitcast_convert_type(alpha - P, jnp.int32)
    half = jnp.int32(0x8000)
    st_ref[0] = (((sbits + half) >> 16) & jnp.int32(0xFFFF)) | \
                ((tbits + half) & jnp.int32(-65536))
    q_ref[0] = varphi + Pb
    pb_ref[0] = Pb


def _prep(xytp, features, W_pos, b_pos, W_lt, b_lt):
    grid = (_B, _N // _BQ)
    return pl.pallas_call(
        _prep_body,
        grid=grid,
        in_specs=[
            pl.BlockSpec((1, _BQ, 4), lambda b, i: (b, i, 0)),
            pl.BlockSpec((1, _BQ, _C), lambda b, i: (b, i, 0)),
            pl.BlockSpec((4, _C), lambda b, i: (0, 0)),
            pl.BlockSpec((1, _C), lambda b, i: (0, 0)),
            pl.BlockSpec((_C, 3 * _C), lambda b, i: (0, 0)),
            pl.BlockSpec((1, 3 * _C), lambda b, i: (0, 0)),
        ],
        out_specs=[
            pl.BlockSpec((1, _BQ, _C), lambda b, i: (b, i, 0)),
            pl.BlockSpec((1, _BQ, _C), lambda b, i: (b, i, 0)),
            pl.BlockSpec((1, _BQ, _C), lambda b, i: (b, i, 0)),
        ],
        out_shape=[
            jax.ShapeDtypeStruct((_B, _N, _C), jnp.int32),
            jax.ShapeDtypeStruct((_B, _N, _C), jnp.float32),
            jax.ShapeDtypeStruct((_B, _N, _C), jnp.float32),
        ],
    )(xytp, features, W_pos, b_pos.reshape(1, _C), W_lt, b_lt.reshape(1, 3 * _C))


def _knn_body(xq_ref, xa_ref, idx_ref, *, base):
    lane4 = lax.broadcasted_iota(jnp.int32, (1, 4), 1)
    xq = jnp.where(lane4 < 3, xq_ref[0], 0.0)
    xa = jnp.where(lane4 < 3, xa_ref[0], 0.0)
    ones14 = jnp.full((1, 4), 1.0, jnp.float32)
    sqa_row = lax.dot_general(ones14, xa * xa, (((1,), (1,)), ((), ())),
                              preferred_element_type=jnp.float32,
                              precision=lax.Precision.HIGHEST)
    dot = lax.dot_general(xq, xa, (((1,), (1,)), ((), ())),
                          preferred_element_type=jnp.float32,
                          precision=lax.Precision.DEFAULT)
    d2 = sqa_row - 2.0 * dot

    kbits = lax.bitcast_convert_type(d2, jnp.int32)
    kmono = kbits ^ ((kbits >> 31) & jnp.int32(0x7FFFFFFF))
    nchunks = _N // 128
    v = [(kmono[:, c * 128:(c + 1) * 128] & jnp.int32(~0x1F)) | jnp.int32(c)
         for c in range(nchunks)]

    def _ce(a, b):
        return jnp.minimum(a, b), jnp.maximum(a, b)

    def _bitonic_sort16(w):
        n = 16
        k = 2
        while k <= n:
            j = k // 2
            while j >= 1:
                for i in range(n):
                    l = i ^ j
                    if l > i:
                        lo, hi = _ce(w[i], w[l])
                        if (i & k) == 0:
                            w[i], w[l] = lo, hi
                        else:
                            w[i], w[l] = hi, lo
                j //= 2
            k *= 2
        return w

    def _bitonic_merge16(w):
        for j in (8, 4, 2, 1):
            for i in range(16):
                l = i ^ j
                if l > i:
                    w[i], w[l] = _ce(w[i], w[l])
        return w

    a = _bitonic_sort16(v[:16])
    bb = _bitonic_sort16(v[16:])
    t16 = [jnp.minimum(a[i], bb[15 - i]) for i in range(16)]
    L = _bitonic_merge16(t16)

    lane = lax.broadcasted_iota(jnp.int32, (_BQ, 128), 1)
    big = jnp.int32(2 ** 30)
    maxi = jnp.int32(0x7FFFFFFF)
    for t in range(_K):
        m = jnp.min(L[0], axis=1, keepdims=True)
        cand = jnp.where(L[0] == m, lane, big)
        l = jnp.min(cand, axis=1, keepdims=True)
        idx_ref[0, :, t:t + 1] = (m & 31) * 128 + l + base
        if t < _K - 1:
            sel = lane == l
            for p in range(15):
                L[p] = jnp.where(sel, L[p + 1], L[p])
            L[15] = jnp.where(sel, maxi, L[15])
    return


def _knn(x_b, base):
    grid = (_N // _BQ,)
    return pl.pallas_call(
        functools.partial(_knn_body, base=base),
        grid=grid,
        in_specs=[
            pl.BlockSpec((1, _BQ, 4), lambda i: (0, i, 0)),
            pl.BlockSpec((1, _N, 4), lambda i: (0, 0, 0)),
        ],
        out_specs=pl.BlockSpec((1, _BQ, _K), lambda i: (0, i, 0)),
        out_shape=jax.ShapeDtypeStruct((1, _N, _K), jnp.int32),
    )(x_b, x_b)


def _sc_gather(table, idxg):
    R, D = table.shape
    M = idxg.shape[0]
    NW = 32
    per_w = M // NW
    CH = 128
    mesh = plsc.VectorSubcoreMesh(core_axis_name="c", subcore_axis_name="s")

    @functools.partial(
        pl.kernel, mesh=mesh,
        out_type=jax.ShapeDtypeStruct((M, D), table.dtype),
        scratch_types=[
            pltpu.VMEM((CH,), jnp.int32),
            pltpu.VMEM((CH, D), table.dtype),
            pltpu.SemaphoreType.DMA,
        ],
    )
    def gather_k(tab_hbm, idx_hbm, out_hbm, idx_v, rows_v, sem):
        c = lax.axis_index("c")
        s = lax.axis_index("s")
        wid = s * 2 + c
        base = wid * per_w

        def body(i, carry):
            off = base + i * CH
            pltpu.sync_copy(idx_hbm.at[pl.ds(off, CH)], idx_v)
            pltpu.async_copy(tab_hbm.at[idx_v], rows_v, sem).wait()
            pltpu.sync_copy(rows_v, out_hbm.at[pl.ds(off, CH)])
            return carry

        lax.fori_loop(0, per_w // CH, body, 0)

    return gather_k(table, idxg)


def _attn_body(q_ref, pb_ref, g_ref, gamma_ref, beta_ref, o_ref):
    Q = q_ref[0]
    Pb = pb_ref[0]
    G = g_ref[0].reshape(_BN, _K, _C)
    S = lax.bitcast_convert_type(G << 16, jnp.float32)
    T = lax.bitcast_convert_type(G & jnp.int32(-65536), jnp.float32)
    pre = Q[:, None, :] - S
    mu = jnp.mean(pre, axis=2, keepdims=True)
    d = pre - mu
    var = jnp.mean(d * d, axis=2, keepdims=True)
    r = 1.0 / jnp.sqrt(var + _EPS)
    ln = d * r * gamma_ref[...] + beta_ref[...]
    z = ln * jnp.float32(1.0 / _SCALE)
    zm = jnp.max(z, axis=1, keepdims=True)
    e = jnp.exp(z - zm)
    w = e * (1.0 / jnp.sum(e, axis=1, keepdims=True))
    o_ref[0] = Pb + jnp.sum(w * T, axis=1)


def _attention(Q, Pb, G, gamma, beta):
    grid = (_N // _BN,)
    return pl.pallas_call(
        _attn_body,
        grid=grid,
        in_specs=[
            pl.BlockSpec((1, _BN, _C), lambda i: (0, i, 0)),
            pl.BlockSpec((1, _BN, _C), lambda i: (0, i, 0)),
            pl.BlockSpec((1, _BN * _K, _C), lambda i: (0, i, 0)),
            pl.BlockSpec((1, _C), lambda i: (0, 0)),
            pl.BlockSpec((1, _C), lambda i: (0, 0)),
        ],
        out_specs=pl.BlockSpec((1, _BN, _C), lambda i: (0, i, 0)),
        out_shape=jax.ShapeDtypeStruct((1, _N, _C), jnp.float32),
    )(Q, Pb, G, gamma.reshape(1, _C), beta.reshape(1, _C))


def kernel(xytp, features, W_pos, b_pos, W_lt, b_lt, gamma, beta):
    ST, Q, Pb = _prep(xytp, features, W_pos, b_pos, W_lt, b_lt)
    ST_flat = ST.reshape(_B * _N, _C)
    outs = []
    for b in range(_B):
        idx_b = _knn(lax.slice_in_dim(xytp, b, b + 1, axis=0), b * _N)
        G_b = _sc_gather(ST_flat, idx_b.reshape(_N * _K))
        outs.append(_attention(
            lax.slice_in_dim(Q, b, b + 1, axis=0),
            lax.slice_in_dim(Pb, b, b + 1, axis=0),
            G_b.reshape(1, _N * _K, _C), gamma, beta))
    return jnp.concatenate(outs, axis=0)


# speedup vs baseline: 1.2417x; 1.0005x over previous
import functools

import jax
import jax.numpy as jnp
import numpy as np
from jax import lax
from jax.experimental import pallas as pl
from jax.experimental.pallas import tpu as pltpu
from jax.experimental.pallas import tpu_sc as plsc

_B, _N, _C, _K = 4, 4096, 128, 16
_EPS = 1e-5
_SCALE = float(np.sqrt(_C))

_BQ = 1024
_BN = 256


def _prep_body(xytp_ref, feat_ref, wpos_ref, bpos_ref, wlt_ref, blt_ref,
               st_ref, q_ref, pb_ref):
    x = xytp_ref[0]
    f = feat_ref[0]
    P = jnp.dot(x, wpos_ref[...], preferred_element_type=jnp.float32,
                precision=lax.Precision.HIGHEST)
    lt = jnp.dot(f, wlt_ref[...], preferred_element_type=jnp.float32,
                 precision=lax.Precision.HIGHEST) + blt_ref[...]
    varphi = lt[:, :_C]
    psi = lt[:, _C:2 * _C]
    alpha = lt[:, 2 * _C:]
    Pb = P + bpos_ref[...]
    sbits = lax.bitcast_convert_type(psi + P, jnp.int32)
    tbits = lax.bitcast_convert_type(alpha - P, jnp.int32)
    half = jnp.int32(0x8000)
    st_ref[0] = (((sbits + half) >> 16) & jnp.int32(0xFFFF)) | \
                ((tbits + half) & jnp.int32(-65536))
    q_ref[0] = varphi + Pb
    pb_ref[0] = Pb


def _prep(xytp, features, W_pos, b_pos, W_lt, b_lt):
    grid = (_B, _N // _BQ)
    return pl.pallas_call(
        _prep_body,
        grid=grid,
        in_specs=[
            pl.BlockSpec((1, _BQ, 4), lambda b, i: (b, i, 0)),
            pl.BlockSpec((1, _BQ, _C), lambda b, i: (b, i, 0)),
            pl.BlockSpec((4, _C), lambda b, i: (0, 0)),
            pl.BlockSpec((1, _C), lambda b, i: (0, 0)),
            pl.BlockSpec((_C, 3 * _C), lambda b, i: (0, 0)),
            pl.BlockSpec((1, 3 * _C), lambda b, i: (0, 0)),
        ],
        out_specs=[
            pl.BlockSpec((1, _BQ, _C), lambda b, i: (b, i, 0)),
            pl.BlockSpec((1, _BQ, _C), lambda b, i: (b, i, 0)),
            pl.BlockSpec((1, _BQ, _C), lambda b, i: (b, i, 0)),
        ],
        out_shape=[
            jax.ShapeDtypeStruct((_B, _N, _C), jnp.int32),
            jax.ShapeDtypeStruct((_B, _N, _C), jnp.float32),
            jax.ShapeDtypeStruct((_B, _N, _C), jnp.float32),
        ],
    )(xytp, features, W_pos, b_pos.reshape(1, _C), W_lt, b_lt.reshape(1, 3 * _C))


def _knn_body(xq_ref, xa_ref, idx_ref, *, base):
    lane4 = lax.broadcasted_iota(jnp.int32, (1, 4), 1)
    xq = jnp.where(lane4 < 3, xq_ref[0], 0.0)
    xa = jnp.where(lane4 < 3, xa_ref[0], 0.0)
    ones14 = jnp.full((1, 4), 1.0, jnp.float32)
    sqa_row = lax.dot_general(ones14, xa * xa, (((1,), (1,)), ((), ())),
                              preferred_element_type=jnp.float32,
                              precision=lax.Precision.HIGHEST)
    dot = lax.dot_general(xq, xa, (((1,), (1,)), ((), ())),
                          preferred_element_type=jnp.float32,
                          precision=lax.Precision.DEFAULT)
    d2 = sqa_row - 2.0 * dot

    kbits = lax.bitcast_convert_type(d2, jnp.int32)
    kmono = kbits ^ ((kbits >> 31) & jnp.int32(0x7FFFFFFF))
    nchunks = _N // 128
    v = [(kmono[:, c * 128:(c + 1) * 128] & jnp.int32(~0x1F)) | jnp.int32(c)
         for c in range(nchunks)]

    def _ce(a, b):
        return jnp.minimum(a, b), jnp.maximum(a, b)

    def _bitonic_sort16(w):
        n = 16
        k = 2
        while k <= n:
            j = k // 2
            while j >= 1:
                for i in range(n):
                    l = i ^ j
                    if l > i:
                        lo, hi = _ce(w[i], w[l])
                        if (i & k) == 0:
                            w[i], w[l] = lo, hi
                        else:
                            w[i], w[l] = hi, lo
                j //= 2
            k *= 2
        return w

    def _bitonic_merge16(w):
        for j in (8, 4, 2, 1):
            for i in range(16):
                l = i ^ j
                if l > i:
                    w[i], w[l] = _ce(w[i], w[l])
        return w

    a = _bitonic_sort16(v[:16])
    bb = _bitonic_sort16(v[16:])
    t16 = [jnp.minimum(a[i], bb[15 - i]) for i in range(16)]
    L = _bitonic_merge16(t16)

    lane = lax.broadcasted_iota(jnp.int32, (_BQ, 128), 1)
    big = jnp.int32(2 ** 30)
    for t in range(_K):
        m = jnp.min(L[0], axis=1, keepdims=True)
        cand = jnp.where(L[0] == m, lane, big)
        l = jnp.min(cand, axis=1, keepdims=True)
        idx_ref[0, :, t:t + 1] = (m & 31) * 128 + l + base
        if t < _K - 1:
            sel = lane == l
            for p in range(15 - t):
                L[p] = jnp.where(sel, L[p + 1], L[p])
    return


def _knn(x_b, base):
    grid = (_N // _BQ,)
    return pl.pallas_call(
        functools.partial(_knn_body, base=base),
        grid=grid,
        in_specs=[
            pl.BlockSpec((1, _BQ, 4), lambda i: (0, i, 0)),
            pl.BlockSpec((1, _N, 4), lambda i: (0, 0, 0)),
        ],
        out_specs=pl.BlockSpec((1, _BQ, _K), lambda i: (0, i, 0)),
        out_shape=jax.ShapeDtypeStruct((1, _N, _K), jnp.int32),
    )(x_b, x_b)


def _sc_gather(table, idxg):
    R, D = table.shape
    M = idxg.shape[0]
    NW = 32
    per_w = M // NW
    CH = 128
    mesh = plsc.VectorSubcoreMesh(core_axis_name="c", subcore_axis_name="s")

    @functools.partial(
        pl.kernel, mesh=mesh,
        out_type=jax.ShapeDtypeStruct((M, D), table.dtype),
        scratch_types=[
            pltpu.VMEM((CH,), jnp.int32),
            pltpu.VMEM((CH, D), table.dtype),
            pltpu.SemaphoreType.DMA,
        ],
    )
    def gather_k(tab_hbm, idx_hbm, out_hbm, idx_v, rows_v, sem):
        c = lax.axis_index("c")
        s = lax.axis_index("s")
        wid = s * 2 + c
        base = wid * per_w

        def body(i, carry):
            off = base + i * CH
            pltpu.sync_copy(idx_hbm.at[pl.ds(off, CH)], idx_v)
            pltpu.async_copy(tab_hbm.at[idx_v], rows_v, sem).wait()
            pltpu.sync_copy(rows_v, out_hbm.at[pl.ds(off, CH)])
            return carry

        lax.fori_loop(0, per_w // CH, body, 0)

    return gather_k(table, idxg)


def _attn_body(q_ref, pb_ref, g_ref, gamma_ref, beta_ref, o_ref):
    Q = q_ref[0]
    Pb = pb_ref[0]
    G = g_ref[0].reshape(_BN, _K, _C)
    S = lax.bitcast_convert_type(G << 16, jnp.float32)
    T = lax.bitcast_convert_type(G & jnp.int32(-65536), jnp.float32)
    pre = Q[:, None, :] - S
    mu = jnp.mean(pre, axis=2, keepdims=True)
    d = pre - mu
    var = jnp.mean(d * d, axis=2, keepdims=True)
    r = 1.0 / jnp.sqrt(var + _EPS)
    ln = d * r * gamma_ref[...] + beta_ref[...]
    z = ln * jnp.float32(1.0 / _SCALE)
    zm = jnp.max(z, axis=1, keepdims=True)
    e = jnp.exp(z - zm)
    w = e * (1.0 / jnp.sum(e, axis=1, keepdims=True))
    o_ref[0] = Pb + jnp.sum(w * T, axis=1)


def _attention(Q, Pb, G, gamma, beta):
    grid = (_N // _BN,)
    return pl.pallas_call(
        _attn_body,
        grid=grid,
        in_specs=[
            pl.BlockSpec((1, _BN, _C), lambda i: (0, i, 0)),
            pl.BlockSpec((1, _BN, _C), lambda i: (0, i, 0)),
            pl.BlockSpec((1, _BN * _K, _C), lambda i: (0, i, 0)),
            pl.BlockSpec((1, _C), lambda i: (0, 0)),
            pl.BlockSpec((1, _C), lambda i: (0, 0)),
        ],
        out_specs=pl.BlockSpec((1, _BN, _C), lambda i: (0, i, 0)),
        out_shape=jax.ShapeDtypeStruct((1, _N, _C), jnp.float32),
    )(Q, Pb, G, gamma.reshape(1, _C), beta.reshape(1, _C))


def kernel(xytp, features, W_pos, b_pos, W_lt, b_lt, gamma, beta):
    ST, Q, Pb = _prep(xytp, features, W_pos, b_pos, W_lt, b_lt)
    ST_flat = ST.reshape(_B * _N, _C)
    outs = []
    for b in range(_B):
        idx_b = _knn(lax.slice_in_dim(xytp, b, b + 1, axis=0), b * _N)
        G_b = _sc_gather(ST_flat, idx_b.reshape(_N * _K))
        outs.append(_attention(
            lax.slice_in_dim(Q, b, b + 1, axis=0),
            lax.slice_in_dim(Pb, b, b + 1, axis=0),
            G_b.reshape(1, _N * _K, _C), gamma, beta))
    return jnp.concatenate(outs, axis=0)
